# baseline (speedup 1.0000x reference)
import functools

import jax
import jax.numpy as jnp
from jax import lax
from jax.experimental import pallas as pl
from jax.experimental.pallas import tpu as pltpu
from jax.experimental.pallas import tpu_sc as plsc

_NC, _NS, _L = 2, 16, 16
_NW = _NC * _NS
_R = 8
_NBUF = 4


def kernel(x, col_idxs):
    m, n = x.shape
    k = col_idxs.shape[0]
    rows_w = m // _NW
    nch = rows_w // _R
    kg = k // _L
    mesh = plsc.VectorSubcoreMesh(core_axis_name="c", subcore_axis_name="s")

    @functools.partial(
        pl.kernel,
        out_type=jax.ShapeDtypeStruct((m, n), x.dtype),
        mesh=mesh,
        compiler_params=pltpu.CompilerParams(
            needs_layout_passes=False,
            use_tc_tiling_on_sc=True,
        ),
        scratch_types=[
            [pltpu.VMEM((_R, n), jnp.float32) for _ in range(_NBUF)],
            pltpu.VMEM((k,), jnp.int32),
            pltpu.SemaphoreType.DMA((_NBUF,)),
            pltpu.SemaphoreType.DMA((_NBUF,)),
        ],
    )
    def sc_kernel(x_hbm, ci_hbm, out_hbm, bufs, ci_v, sin, sout):
        wid = lax.axis_index("s") * _NC + lax.axis_index("c")
        base = wid * rows_w

        pltpu.sync_copy(ci_hbm, ci_v)
        cvs = [ci_v[pl.ds(j * _L, _L)] for j in range(kg)]
        ridxs = [jnp.full((_L,), r, jnp.int32) for r in range(_R)]

        def in_copy(g, b):
            return pltpu.make_async_copy(
                x_hbm.at[pl.ds(base + g * _R, _R), :], bufs[b], sin.at[b]
            )

        def out_copy(g, b):
            return pltpu.make_async_copy(
                bufs[b], out_hbm.at[pl.ds(base + g * _R, _R), :], sout.at[b]
            )

        def fix(b):
            for r in range(_R):
                for j in range(kg):
                    v = plsc.load_gather(bufs[b], [ridxs[r], cvs[j]])
                    bv = jnp.where(v >= 0.0, 1.0, 0.0).astype(v.dtype)
                    plsc.store_scatter(bufs[b], [ridxs[r], cvs[j]], bv)

        for b in range(_NBUF - 1):
            in_copy(b, b).start()

        def outer(t, carry):
            c0 = t * _NBUF
            for b in range(_NBUF):
                g = c0 + b
                in_copy(g, b).wait()
                fix(b)
                out_copy(g, b).start()
                p = g + _NBUF - 1
                pb = (b + _NBUF - 1) % _NBUF

                @pl.when(jnp.logical_and(g >= 1, p < nch))
                def _wait_prev():
                    out_copy(g - 1, pb).wait()

                @pl.when(p < nch)
                def _prefetch():
                    in_copy(p, pb).start()

            return carry

        lax.fori_loop(0, nch // _NBUF, outer, 0)

        for b in range(_NBUF):
            out_copy(nch - _NBUF + b, b).wait()

    return sc_kernel(x, col_idxs)


# speedup vs baseline: 1.0133x; 1.0133x over previous
import functools

import jax
import jax.numpy as jnp
from jax import lax
from jax.experimental import pallas as pl
from jax.experimental.pallas import tpu as pltpu
from jax.experimental.pallas import tpu_sc as plsc

_NC, _NS, _L = 2, 16, 16
_NW = _NC * _NS
_R = 4
_NBUF = 8


def kernel(x, col_idxs):
    m, n = x.shape
    k = col_idxs.shape[0]
    rows_w = m // _NW
    nch = rows_w // _R
    kg = k // _L
    mesh = plsc.VectorSubcoreMesh(core_axis_name="c", subcore_axis_name="s")

    @functools.partial(
        pl.kernel,
        out_type=jax.ShapeDtypeStruct((m, n), x.dtype),
        mesh=mesh,
        compiler_params=pltpu.CompilerParams(
            needs_layout_passes=False,
            use_tc_tiling_on_sc=True,
        ),
        scratch_types=[
            [pltpu.VMEM((_R, n), jnp.float32) for _ in range(_NBUF)],
            pltpu.VMEM((k,), jnp.int32),
            pltpu.SemaphoreType.DMA((_NBUF,)),
            pltpu.SemaphoreType.DMA((_NBUF,)),
        ],
    )
    def sc_kernel(x_hbm, ci_hbm, out_hbm, bufs, ci_v, sin, sout):
        wid = lax.axis_index("s") * _NC + lax.axis_index("c")
        base = wid * rows_w

        pltpu.sync_copy(ci_hbm, ci_v)
        cvs = [ci_v[pl.ds(j * _L, _L)] for j in range(kg)]
        ridxs = [jnp.full((_L,), r, jnp.int32) for r in range(_R)]

        def in_copy(g, b):
            return pltpu.make_async_copy(
                x_hbm.at[pl.ds(base + g * _R, _R), :], bufs[b], sin.at[b]
            )

        def out_copy(g, b):
            return pltpu.make_async_copy(
                bufs[b], out_hbm.at[pl.ds(base + g * _R, _R), :], sout.at[b]
            )

        def fix(b):
            for r in range(_R):
                for j in range(kg):
                    v = plsc.load_gather(bufs[b], [ridxs[r], cvs[j]])
                    bv = jnp.where(v >= 0.0, 1.0, 0.0).astype(v.dtype)
                    plsc.store_scatter(bufs[b], [ridxs[r], cvs[j]], bv)

        for b in range(_NBUF - 1):
            in_copy(b, b).start()

        def outer(t, carry):
            c0 = t * _NBUF
            for b in range(_NBUF):
                g = c0 + b
                in_copy(g, b).wait()
                fix(b)
                out_copy(g, b).start()
                p = g + _NBUF - 1
                pb = (b + _NBUF - 1) % _NBUF

                @pl.when(jnp.logical_and(g >= 1, p < nch))
                def _wait_prev():
                    out_copy(g - 1, pb).wait()

                @pl.when(p < nch)
                def _prefetch():
                    in_copy(p, pb).start()

            return carry

        lax.fori_loop(0, nch // _NBUF, outer, 0)

        for b in range(_NBUF):
            out_copy(nch - _NBUF + b, b).wait()

    return sc_kernel(x, col_idxs)
